# speedup vs baseline: 1.0083x; 1.0083x over previous
import jax
import jax.numpy as jnp
from jax.experimental import pallas as pl
from jax.experimental.pallas import tpu as pltpu


def _bilinear_matrix(n_in: int) -> jnp.ndarray:
    n_out = 2 * n_in
    src = (jnp.arange(n_out, dtype=jnp.float32) + 0.5) * 0.5 - 0.5
    src = jnp.maximum(src, 0.0)
    i0 = jnp.minimum(jnp.floor(src).astype(jnp.int32), n_in - 1)
    i1 = jnp.minimum(i0 + 1, n_in - 1)
    l1 = src - i0.astype(jnp.float32)
    l0 = 1.0 - l1
    rows = jnp.arange(n_out)
    u = jnp.zeros((n_out, n_in), jnp.float32)
    u = u.at[rows, i0].add(l0)
    u = u.at[rows, i1].add(l1)
    return u


def _su_kernel(x_ref, w_ref, u_ref, skip_ref, o_ref):
    xb = x_ref[0].astype(jnp.bfloat16)
    z = jnp.dot(w_ref[...], xb, preferred_element_type=jnp.float32)
    y = jnp.dot(z.astype(jnp.bfloat16), u_ref[...],
                preferred_element_type=jnp.float32)
    o_ref[0] = y + skip_ref[0]


@jax.jit
def kernel(x_nchw, skip_nchw, conv_weight):
    n, cin, h, w = x_nchw.shape
    cout = conv_weight.shape[0]
    hw = h * w
    dtype = x_nchw.dtype

    wmat = conv_weight.reshape(cout, cin).astype(jnp.bfloat16)
    uh = _bilinear_matrix(h)
    uw = _bilinear_matrix(w)
    u = jnp.kron(uh.T, uw.T).astype(jnp.bfloat16)

    x_flat = x_nchw.reshape(n, cin, hw)
    skip_flat = skip_nchw.reshape(n, cout, 4 * hw)

    flops = 2 * n * cout * cin * hw + 2 * n * cout * hw * 4 * hw + n * cout * 4 * hw
    bytes_accessed = 4 * (n * cin * hw + 2 * n * cout * 4 * hw) + 2 * (
        cout * cin + hw * 4 * hw)

    out_flat = pl.pallas_call(
        _su_kernel,
        out_shape=jax.ShapeDtypeStruct((n, cout, 4 * hw), dtype),
        grid=(n,),
        in_specs=[
            pl.BlockSpec((1, cin, hw), lambda i: (i, 0, 0)),
            pl.BlockSpec((cout, cin), lambda i: (0, 0)),
            pl.BlockSpec((hw, 4 * hw), lambda i: (0, 0)),
            pl.BlockSpec((1, cout, 4 * hw), lambda i: (i, 0, 0)),
        ],
        out_specs=pl.BlockSpec((1, cout, 4 * hw), lambda i: (i, 0, 0)),
        compiler_params=pltpu.CompilerParams(
            dimension_semantics=("parallel",),
            vmem_limit_bytes=48 * 1024 * 1024),
        cost_estimate=pl.CostEstimate(flops=int(flops), transcendentals=0,
                                      bytes_accessed=int(bytes_accessed)),
    )(x_flat, wmat, u, skip_flat)
    return out_flat.reshape(n, cout, 2 * h, 2 * w)


# speedup vs baseline: 1.1698x; 1.1602x over previous
import functools

import numpy as np

import jax
import jax.numpy as jnp
from jax.experimental import pallas as pl
from jax.experimental.pallas import tpu as pltpu


def _bilinear_matrix(n_in: int) -> np.ndarray:
    n_out = 2 * n_in
    src = (np.arange(n_out, dtype=np.float64) + 0.5) * 0.5 - 0.5
    src = np.maximum(src, 0.0)
    i0 = np.minimum(np.floor(src).astype(np.int64), n_in - 1)
    i1 = np.minimum(i0 + 1, n_in - 1)
    l1 = (src - i0).astype(np.float32)
    l0 = 1.0 - l1
    rows = np.arange(n_out)
    u = np.zeros((n_out, n_in), np.float32)
    np.add.at(u, (rows, i0), l0)
    np.add.at(u, (rows, i1), l1)
    return u


@functools.lru_cache(maxsize=None)
def _kron_u(h: int, w: int) -> np.ndarray:
    return np.kron(_bilinear_matrix(h).T, _bilinear_matrix(w).T).astype(
        jnp.bfloat16)


def _su_kernel(x_ref, w_ref, u_ref, skip_ref, o_ref):
    xb = x_ref[0].astype(jnp.bfloat16)
    z = jnp.dot(w_ref[...], xb, preferred_element_type=jnp.float32)
    y = jnp.dot(z.astype(jnp.bfloat16), u_ref[...],
                preferred_element_type=jnp.float32)
    o_ref[0] = y + skip_ref[0]


@jax.jit
def kernel(x_nchw, skip_nchw, conv_weight):
    n, cin, h, w = x_nchw.shape
    cout = conv_weight.shape[0]
    hw = h * w
    dtype = x_nchw.dtype

    wmat = conv_weight.reshape(cout, cin).astype(jnp.bfloat16)
    u = jnp.asarray(_kron_u(h, w))

    x_flat = x_nchw.reshape(n, cin, hw)
    skip_flat = skip_nchw.reshape(n, cout, 4 * hw)

    flops = 2 * n * cout * cin * hw + 2 * n * cout * hw * 4 * hw + n * cout * 4 * hw
    bytes_accessed = 4 * (n * cin * hw + 2 * n * cout * 4 * hw) + 2 * (
        cout * cin + hw * 4 * hw)

    out_flat = pl.pallas_call(
        _su_kernel,
        out_shape=jax.ShapeDtypeStruct((n, cout, 4 * hw), dtype),
        grid=(n,),
        in_specs=[
            pl.BlockSpec((1, cin, hw), lambda i: (i, 0, 0)),
            pl.BlockSpec((cout, cin), lambda i: (0, 0)),
            pl.BlockSpec((hw, 4 * hw), lambda i: (0, 0)),
            pl.BlockSpec((1, cout, 4 * hw), lambda i: (i, 0, 0)),
        ],
        out_specs=pl.BlockSpec((1, cout, 4 * hw), lambda i: (i, 0, 0)),
        compiler_params=pltpu.CompilerParams(
            dimension_semantics=("parallel",),
            vmem_limit_bytes=48 * 1024 * 1024),
        cost_estimate=pl.CostEstimate(flops=int(flops), transcendentals=0,
                                      bytes_accessed=int(bytes_accessed)),
    )(x_flat, wmat, u, skip_flat)
    return out_flat.reshape(n, cout, 2 * h, 2 * w)


# speedup vs baseline: 1.3303x; 1.1372x over previous
import functools

import numpy as np

import jax
import jax.numpy as jnp
from jax.experimental import pallas as pl
from jax.experimental.pallas import tpu as pltpu


def _bilinear_matrix(n_in: int) -> np.ndarray:
    n_out = 2 * n_in
    src = (np.arange(n_out, dtype=np.float64) + 0.5) * 0.5 - 0.5
    src = np.maximum(src, 0.0)
    i0 = np.minimum(np.floor(src).astype(np.int64), n_in - 1)
    i1 = np.minimum(i0 + 1, n_in - 1)
    l1 = (src - i0).astype(np.float32)
    l0 = 1.0 - l1
    rows = np.arange(n_out)
    u = np.zeros((n_out, n_in), np.float32)
    np.add.at(u, (rows, i0), l0)
    np.add.at(u, (rows, i1), l1)
    return u


@functools.lru_cache(maxsize=None)
def _kron_u(h: int, w: int) -> np.ndarray:
    return np.kron(_bilinear_matrix(h).T, _bilinear_matrix(w).T).astype(
        jnp.bfloat16)


def _su_kernel(x_ref, w_ref, u_ref, skip_ref, o_ref, *, bn):
    for b in range(bn):
        xb = x_ref[b].astype(jnp.bfloat16)
        z = jnp.dot(w_ref[...], xb, preferred_element_type=jnp.float32)
        y = jnp.dot(z.astype(jnp.bfloat16), u_ref[...],
                    preferred_element_type=jnp.float32)
        o_ref[b] = y + skip_ref[b]


@jax.jit
def kernel(x_nchw, skip_nchw, conv_weight):
    n, cin, h, w = x_nchw.shape
    cout = conv_weight.shape[0]
    hw = h * w
    dtype = x_nchw.dtype

    wmat = conv_weight.reshape(cout, cin).astype(jnp.bfloat16)
    u = jnp.asarray(_kron_u(h, w))

    x_flat = x_nchw.reshape(n, cin, hw)
    skip_flat = skip_nchw.reshape(n, cout, 4 * hw)

    flops = 2 * n * cout * cin * hw + 2 * n * cout * hw * 4 * hw + n * cout * 4 * hw
    bytes_accessed = 4 * (n * cin * hw + 2 * n * cout * 4 * hw) + 2 * (
        cout * cin + hw * 4 * hw)

    bn = 4 if n % 4 == 0 else 1
    out_flat = pl.pallas_call(
        functools.partial(_su_kernel, bn=bn),
        out_shape=jax.ShapeDtypeStruct((n, cout, 4 * hw), dtype),
        grid=(n // bn,),
        in_specs=[
            pl.BlockSpec((bn, cin, hw), lambda i: (i, 0, 0)),
            pl.BlockSpec((cout, cin), lambda i: (0, 0)),
            pl.BlockSpec((hw, 4 * hw), lambda i: (0, 0)),
            pl.BlockSpec((bn, cout, 4 * hw), lambda i: (i, 0, 0)),
        ],
        out_specs=pl.BlockSpec((bn, cout, 4 * hw), lambda i: (i, 0, 0)),
        compiler_params=pltpu.CompilerParams(
            dimension_semantics=("parallel",),
            vmem_limit_bytes=48 * 1024 * 1024),
        cost_estimate=pl.CostEstimate(flops=int(flops), transcendentals=0,
                                      bytes_accessed=int(bytes_accessed)),
    )(x_flat, wmat, u, skip_flat)
    return out_flat.reshape(n, cout, 2 * h, 2 * w)


# speedup vs baseline: 1.3409x; 1.0080x over previous
import functools

import numpy as np

import jax
import jax.numpy as jnp
from jax.experimental import pallas as pl
from jax.experimental.pallas import tpu as pltpu


def _bilinear_matrix(n_in: int) -> np.ndarray:
    n_out = 2 * n_in
    src = (np.arange(n_out, dtype=np.float64) + 0.5) * 0.5 - 0.5
    src = np.maximum(src, 0.0)
    i0 = np.minimum(np.floor(src).astype(np.int64), n_in - 1)
    i1 = np.minimum(i0 + 1, n_in - 1)
    l1 = (src - i0).astype(np.float32)
    l0 = 1.0 - l1
    rows = np.arange(n_out)
    u = np.zeros((n_out, n_in), np.float32)
    np.add.at(u, (rows, i0), l0)
    np.add.at(u, (rows, i1), l1)
    return u


@functools.lru_cache(maxsize=None)
def _kron_u(h: int, w: int) -> np.ndarray:
    return np.kron(_bilinear_matrix(h).T, _bilinear_matrix(w).T).astype(
        jnp.bfloat16)


def _su_kernel(x_ref, w_ref, u_ref, skip_ref, o_ref, *, bn):
    for b in range(bn):
        xb = x_ref[b].astype(jnp.bfloat16)
        z = jnp.dot(w_ref[...], xb, preferred_element_type=jnp.float32)
        y = jnp.dot(z.astype(jnp.bfloat16), u_ref[...],
                    preferred_element_type=jnp.float32)
        o_ref[b] = y + skip_ref[b]


@jax.jit
def kernel(x_nchw, skip_nchw, conv_weight):
    n, cin, h, w = x_nchw.shape
    cout = conv_weight.shape[0]
    hw = h * w
    dtype = x_nchw.dtype

    wmat = conv_weight.reshape(cout, cin).astype(jnp.bfloat16)
    u = jnp.asarray(_kron_u(h, w))

    x_flat = x_nchw.reshape(n, cin, hw)
    skip_flat = skip_nchw.reshape(n, cout, 4 * hw)

    flops = 2 * n * cout * cin * hw + 2 * n * cout * hw * 4 * hw + n * cout * 4 * hw
    bytes_accessed = 4 * (n * cin * hw + 2 * n * cout * 4 * hw) + 2 * (
        cout * cin + hw * 4 * hw)

    bn = 8 if n % 8 == 0 else 1
    out_flat = pl.pallas_call(
        functools.partial(_su_kernel, bn=bn),
        out_shape=jax.ShapeDtypeStruct((n, cout, 4 * hw), dtype),
        grid=(n // bn,),
        in_specs=[
            pl.BlockSpec((bn, cin, hw), lambda i: (i, 0, 0)),
            pl.BlockSpec((cout, cin), lambda i: (0, 0)),
            pl.BlockSpec((hw, 4 * hw), lambda i: (0, 0)),
            pl.BlockSpec((bn, cout, 4 * hw), lambda i: (i, 0, 0)),
        ],
        out_specs=pl.BlockSpec((bn, cout, 4 * hw), lambda i: (i, 0, 0)),
        compiler_params=pltpu.CompilerParams(
            dimension_semantics=("parallel",),
            vmem_limit_bytes=100 * 1024 * 1024),
        cost_estimate=pl.CostEstimate(flops=int(flops), transcendentals=0,
                                      bytes_accessed=int(bytes_accessed)),
    )(x_flat, wmat, u, skip_flat)
    return out_flat.reshape(n, cout, 2 * h, 2 * w)
